# Initial kernel scaffold; baseline (speedup 1.0000x reference)
import jax
import jax.numpy as jnp
from jax.experimental import pallas as pl


def kernel(action_indices, table, W1, b1, W2, b2):
    raise NotImplementedError("write your pallas kernel here")


# speedup vs baseline: 16.8035x; 16.8035x over previous
import functools

import jax
import jax.numpy as jnp
from jax import lax
from jax.experimental import pallas as pl
from jax.experimental.pallas import tpu as pltpu
from jax.experimental.pallas import tpu_sc as plsc

NUM_ACTIONS = 100000
EMBED_DIM = 64
HIDDEN_DIM = 256
BATCH = 16384
SEQ = 50

ROW_BLOCK = 1000

NC = 2
NS = 16
NW = NC * NS
CHUNK = 128
TOTAL = BATCH * SEQ
N_CHUNKS = TOTAL // CHUNK
CHUNKS_PER_W = N_CHUNKS // NW


def _mlp_block(table_ref, w1_ref, b1_ref, w2_ref, b2_ref, out_ref):
    t = table_ref[...]
    h = jnp.dot(t, w1_ref[...], preferred_element_type=jnp.float32) + b1_ref[...]
    h = h * jax.nn.sigmoid(h)
    out_ref[...] = (
        jnp.dot(h, w2_ref[...], preferred_element_type=jnp.float32) + b2_ref[...]
    )


def _fuse_table(table, W1, b1, W2, b2):
    grid = (NUM_ACTIONS // ROW_BLOCK,)
    return pl.pallas_call(
        _mlp_block,
        grid=grid,
        in_specs=[
            pl.BlockSpec((ROW_BLOCK, HIDDEN_DIM), lambda i: (i, 0)),
            pl.BlockSpec((HIDDEN_DIM, HIDDEN_DIM), lambda i: (0, 0)),
            pl.BlockSpec((1, HIDDEN_DIM), lambda i: (0, 0)),
            pl.BlockSpec((HIDDEN_DIM, EMBED_DIM), lambda i: (0, 0)),
            pl.BlockSpec((1, EMBED_DIM), lambda i: (0, 0)),
        ],
        out_specs=pl.BlockSpec((ROW_BLOCK, EMBED_DIM), lambda i: (i, 0)),
        out_shape=jax.ShapeDtypeStruct((NUM_ACTIONS, EMBED_DIM), jnp.float32),
    )(table, W1, b1.reshape(1, HIDDEN_DIM), W2, b2.reshape(1, EMBED_DIM))


def _gather_body(fused_hbm, idx_hbm, out_hbm, idx_v, rows_v, sem):
    wid = lax.axis_index("s") * NC + lax.axis_index("c")
    base = wid * CHUNKS_PER_W
    pltpu.sync_copy(idx_hbm.at[pl.ds(base, CHUNKS_PER_W)], idx_v)

    def step(j, _):
        pltpu.async_copy(fused_hbm.at[idx_v.at[j]], rows_v, sem).wait()
        pltpu.sync_copy(rows_v, out_hbm.at[pl.ds((base + j) * CHUNK, CHUNK)])
        return 0

    lax.fori_loop(0, CHUNKS_PER_W, step, 0)


@jax.jit
def _sc_gather(fused, idx2d):
    mesh = plsc.VectorSubcoreMesh(core_axis_name="c", subcore_axis_name="s")
    return pl.kernel(
        _gather_body,
        out_type=jax.ShapeDtypeStruct((TOTAL, EMBED_DIM), jnp.float32),
        mesh=mesh,
        compiler_params=pltpu.CompilerParams(use_tc_tiling_on_sc=False),
        scratch_types=[
            pltpu.VMEM((CHUNKS_PER_W, CHUNK), jnp.int32),
            pltpu.VMEM((CHUNK, EMBED_DIM), jnp.float32),
            pltpu.SemaphoreType.DMA,
        ],
    )(fused, idx2d)


def kernel(action_indices, table, W1, b1, W2, b2):
    idx2d = action_indices.reshape(-1).astype(jnp.int32).reshape(N_CHUNKS, CHUNK)
    fused = _fuse_table(table, W1, b1, W2, b2)
    flat = _sc_gather(fused, idx2d)
    return flat.reshape(BATCH, SEQ, EMBED_DIM)


# speedup vs baseline: 18.7142x; 1.1137x over previous
import functools

import jax
import jax.numpy as jnp
from jax import lax
from jax.experimental import pallas as pl
from jax.experimental.pallas import tpu as pltpu
from jax.experimental.pallas import tpu_sc as plsc

NUM_ACTIONS = 100000
EMBED_DIM = 64
HIDDEN_DIM = 256
BATCH = 16384
SEQ = 50

ROW_BLOCK = 1000

NC = 2
NS = 16
NW = NC * NS
TOTAL = BATCH * SEQ
ROWS_PER_W = BATCH // NW
GROUP = 8
GROUPS_PER_W = ROWS_PER_W // GROUP


def _mlp_block(table_ref, w1_ref, b1_ref, w2_ref, b2_ref, out_ref):
    t = table_ref[...]
    h = jnp.dot(t, w1_ref[...], preferred_element_type=jnp.float32) + b1_ref[...]
    h = h * jax.nn.sigmoid(h)
    out_ref[...] = (
        jnp.dot(h, w2_ref[...], preferred_element_type=jnp.float32) + b2_ref[...]
    )


def _fuse_table(table, W1, b1, W2, b2):
    grid = (NUM_ACTIONS // ROW_BLOCK,)
    return pl.pallas_call(
        _mlp_block,
        grid=grid,
        in_specs=[
            pl.BlockSpec((ROW_BLOCK, HIDDEN_DIM), lambda i: (i, 0)),
            pl.BlockSpec((HIDDEN_DIM, HIDDEN_DIM), lambda i: (0, 0)),
            pl.BlockSpec((1, HIDDEN_DIM), lambda i: (0, 0)),
            pl.BlockSpec((HIDDEN_DIM, EMBED_DIM), lambda i: (0, 0)),
            pl.BlockSpec((1, EMBED_DIM), lambda i: (0, 0)),
        ],
        out_specs=pl.BlockSpec((ROW_BLOCK, EMBED_DIM), lambda i: (i, 0)),
        out_shape=jax.ShapeDtypeStruct((NUM_ACTIONS, EMBED_DIM), jnp.float32),
    )(table, W1, b1.reshape(1, HIDDEN_DIM), W2, b2.reshape(1, EMBED_DIM))


def _gather_body(fused_hbm, idx_hbm, out_hbm, idx_v, rows_v, sem):
    wid = lax.axis_index("s") * NC + lax.axis_index("c")
    base = wid * ROWS_PER_W
    pltpu.sync_copy(idx_hbm.at[pl.ds(base, ROWS_PER_W)], idx_v)

    def step(g, _):
        copies = [
            pltpu.async_copy(
                fused_hbm.at[idx_v.at[g * GROUP + k]], rows_v.at[k], sem
            )
            for k in range(GROUP)
        ]
        for c in copies:
            c.wait()
        pltpu.sync_copy(rows_v, out_hbm.at[pl.ds(base + g * GROUP, GROUP)])
        return 0

    lax.fori_loop(0, GROUPS_PER_W, step, 0)


@jax.jit
def _sc_gather(fused, idx2d):
    mesh = plsc.VectorSubcoreMesh(core_axis_name="c", subcore_axis_name="s")
    return pl.kernel(
        _gather_body,
        out_type=jax.ShapeDtypeStruct((BATCH, SEQ, EMBED_DIM), jnp.float32),
        mesh=mesh,
        compiler_params=pltpu.CompilerParams(use_tc_tiling_on_sc=False),
        scratch_types=[
            pltpu.VMEM((ROWS_PER_W, SEQ), jnp.int32),
            pltpu.VMEM((GROUP, SEQ, EMBED_DIM), jnp.float32),
            pltpu.SemaphoreType.DMA,
        ],
    )(fused, idx2d)


def kernel(action_indices, table, W1, b1, W2, b2):
    idx2d = action_indices.astype(jnp.int32)
    fused = _fuse_table(table, W1, b1, W2, b2)
    return _sc_gather(fused, idx2d)


# speedup vs baseline: 19.5854x; 1.0466x over previous
import functools

import jax
import jax.numpy as jnp
from jax import lax
from jax.experimental import pallas as pl
from jax.experimental.pallas import tpu as pltpu
from jax.experimental.pallas import tpu_sc as plsc

NUM_ACTIONS = 100000
EMBED_DIM = 64
HIDDEN_DIM = 256
BATCH = 16384
SEQ = 50

ROW_BLOCK = 1000

NC = 2
NS = 16
NW = NC * NS
TOTAL = BATCH * SEQ
ROWS_PER_W = BATCH // NW
GROUP = 8
GROUPS_PER_W = ROWS_PER_W // GROUP


def _mlp_block(table_ref, w1_ref, b1_ref, w2_ref, b2_ref, out_ref):
    t = table_ref[...]
    h = jnp.dot(t, w1_ref[...], preferred_element_type=jnp.float32) + b1_ref[...]
    h = h * jax.nn.sigmoid(h)
    out_ref[...] = (
        jnp.dot(h, w2_ref[...], preferred_element_type=jnp.float32) + b2_ref[...]
    )


def _fuse_table(table, W1, b1, W2, b2):
    grid = (NUM_ACTIONS // ROW_BLOCK,)
    return pl.pallas_call(
        _mlp_block,
        grid=grid,
        in_specs=[
            pl.BlockSpec((ROW_BLOCK, HIDDEN_DIM), lambda i: (i, 0)),
            pl.BlockSpec((HIDDEN_DIM, HIDDEN_DIM), lambda i: (0, 0)),
            pl.BlockSpec((1, HIDDEN_DIM), lambda i: (0, 0)),
            pl.BlockSpec((HIDDEN_DIM, EMBED_DIM), lambda i: (0, 0)),
            pl.BlockSpec((1, EMBED_DIM), lambda i: (0, 0)),
        ],
        out_specs=pl.BlockSpec((ROW_BLOCK, EMBED_DIM), lambda i: (i, 0)),
        out_shape=jax.ShapeDtypeStruct((NUM_ACTIONS, EMBED_DIM), jnp.float32),
    )(table, W1, b1.reshape(1, HIDDEN_DIM), W2, b2.reshape(1, EMBED_DIM))


def _gather_body(fused_hbm, idx_hbm, out_hbm, idx_v, rows_v, sem_a, sem_b):
    wid = lax.axis_index("s") * NC + lax.axis_index("c")
    base = wid * ROWS_PER_W
    pltpu.sync_copy(idx_hbm.at[pl.ds(base, ROWS_PER_W)], idx_v)

    def fire(g, slot, sem):
        for k in range(GROUP):
            pltpu.async_copy(
                fused_hbm.at[idx_v.at[g * GROUP + k]], rows_v.at[slot, k], sem
            )

    def drain_and_write(g, slot, sem):
        for k in range(GROUP):
            pltpu.make_async_copy(
                fused_hbm.at[idx_v.at[k]], rows_v.at[slot, k], sem
            ).wait()
        pltpu.sync_copy(rows_v.at[slot], out_hbm.at[pl.ds(base + g * GROUP, GROUP)])

    fire(0, 0, sem_a)

    def step(g, _):
        even = lax.rem(g, 2) == 0
        more = g + 1 < GROUPS_PER_W

        @pl.when(jnp.logical_and(even, more))
        def _():
            fire(g + 1, 1, sem_b)

        @pl.when(jnp.logical_and(jnp.logical_not(even), more))
        def _():
            fire(g + 1, 0, sem_a)

        @pl.when(even)
        def _():
            drain_and_write(g, 0, sem_a)

        @pl.when(jnp.logical_not(even))
        def _():
            drain_and_write(g, 1, sem_b)

        return 0

    lax.fori_loop(0, GROUPS_PER_W, step, 0)


@jax.jit
def _sc_gather(fused, idx2d):
    mesh = plsc.VectorSubcoreMesh(core_axis_name="c", subcore_axis_name="s")
    return pl.kernel(
        _gather_body,
        out_type=jax.ShapeDtypeStruct((BATCH, SEQ, EMBED_DIM), jnp.float32),
        mesh=mesh,
        compiler_params=pltpu.CompilerParams(use_tc_tiling_on_sc=False),
        scratch_types=[
            pltpu.VMEM((ROWS_PER_W, SEQ), jnp.int32),
            pltpu.VMEM((2, GROUP, SEQ, EMBED_DIM), jnp.float32),
            pltpu.SemaphoreType.DMA,
            pltpu.SemaphoreType.DMA,
        ],
    )(fused, idx2d)


def kernel(action_indices, table, W1, b1, W2, b2):
    idx2d = action_indices.astype(jnp.int32)
    fused = _fuse_table(table, W1, b1, W2, b2)
    return _sc_gather(fused, idx2d)
